# baseline (speedup 1.0000x reference)
import jax
import jax.numpy as jnp
from jax import lax
from jax.experimental import pallas as pl

B = 16384
HN = 8
OBS = HN + 1
ANUM = 2 ** HN
VQ_DIM = 4
VQ_SIZE = 8
LAT = 16
NN0, NN1 = 258, 128

BB = 2048


def _vq_block(o_ref, a_ref, w1o_ref, w1a_ref, b1_ref, w2_ref, b2_ref,
              wl_ref, bl_ref, cb_ref, a1v_ref, a1o_ref, ab1_ref,
              a2_ref, ab2_ref, ap_ref, abp_ref, o1_ref, ob1_ref,
              o2_ref, ob2_ref, oo_ref, obo_ref,
              rep_ref, reo_ref, lat_ref, vq_ref, q_ref):
    f32 = jnp.float32
    xo = o_ref[...]
    a = a_ref[...]

    iota_a = lax.broadcasted_iota(jnp.int32, (BB, ANUM), 1)
    a_hot = (iota_a == a).astype(f32)

    h = xo @ w1o_ref[...] + a_hot @ w1a_ref[...] + b1_ref[...]
    h = jnp.maximum(h, 0.0)
    h = jnp.maximum(h @ w2_ref[...] + b2_ref[...], 0.0)
    lat = h @ wl_ref[...] + bl_ref[...]
    lat_ref[...] = lat

    cb = cb_ref[...]
    cbsq = jnp.sum(cb * cb, axis=1)[None, :]
    iota_c = lax.broadcasted_iota(jnp.int32, (BB, VQ_SIZE), 1)
    q_parts = []
    for g in range(LAT // VQ_DIM):
        lg = lat[:, g * VQ_DIM:(g + 1) * VQ_DIM]
        dist = (jnp.sum(lg * lg, axis=1, keepdims=True) + cbsq
                - 2.0 * (lg @ cb.T))
        m = jnp.min(dist, axis=1, keepdims=True)
        idx = jnp.min(jnp.where(dist == m, iota_c, VQ_SIZE),
                      axis=1, keepdims=True)
        onehot = (iota_c == idx).astype(f32)
        q_parts.append(onehot @ cb)
    q = jnp.concatenate(q_parts, axis=1)
    vq = lat + (q - lat)
    vq_ref[...] = vq
    q_ref[...] = q

    ha = vq @ a1v_ref[...] + xo @ a1o_ref[...] + ab1_ref[...]
    ha = jnp.maximum(ha, 0.0)
    ha = jnp.maximum(ha @ a2_ref[...] + ab2_ref[...], 0.0)
    rep_ref[...] = ha @ ap_ref[...] + abp_ref[...]

    ho = jnp.maximum(vq @ o1_ref[...] + ob1_ref[...], 0.0)
    ho = jnp.maximum(ho @ o2_ref[...] + ob2_ref[...], 0.0)
    reo_ref[...] = ho @ oo_ref[...] + obo_ref[...]


def kernel(o, a, enc_w1, enc_b1, enc_w2, enc_b2, enc_wl, enc_bl, codebook,
           ad_w1, ad_b1, ad_w2, ad_b2, ad_wp, ad_bp,
           od_w1, od_b1, od_w2, od_b2, od_wo, od_bo):
    f32 = jnp.float32
    w1o = enc_w1[:, :OBS].T
    w1a = enc_w1[:, OBS:].T
    w2 = enc_w2.T
    wl = enc_wl.T
    a1v = ad_w1[:, :LAT].T
    a1o = ad_w1[:, LAT:].T
    a2 = ad_w2.T
    ap = ad_wp.T
    o1 = od_w1.T
    o2 = od_w2.T
    oo = od_wo.T

    def b2d(v):
        return v.reshape(1, -1)

    grid = (B // BB,)

    def bcast_spec(arr):
        return pl.BlockSpec(arr.shape, lambda i: (0, 0))

    weights = [w1o, w1a, b2d(enc_b1), w2, b2d(enc_b2), wl, b2d(enc_bl),
               codebook, a1v, a1o, b2d(ad_b1), a2, b2d(ad_b2), ap,
               b2d(ad_bp), o1, b2d(od_b1), o2, b2d(od_b2), oo, b2d(od_bo)]

    in_specs = [
        pl.BlockSpec((BB, OBS), lambda i: (i, 0)),
        pl.BlockSpec((BB, 1), lambda i: (i, 0)),
    ] + [bcast_spec(w) for w in weights]

    out_specs = (
        pl.BlockSpec((BB, ANUM), lambda i: (i, 0)),
        pl.BlockSpec((BB, OBS), lambda i: (i, 0)),
        pl.BlockSpec((BB, LAT), lambda i: (i, 0)),
        pl.BlockSpec((BB, LAT), lambda i: (i, 0)),
        pl.BlockSpec((BB, LAT), lambda i: (i, 0)),
    )
    out_shape = (
        jax.ShapeDtypeStruct((B, ANUM), f32),
        jax.ShapeDtypeStruct((B, OBS), f32),
        jax.ShapeDtypeStruct((B, LAT), f32),
        jax.ShapeDtypeStruct((B, LAT), f32),
        jax.ShapeDtypeStruct((B, LAT), f32),
    )

    re_p, re_o, latent, vq_latent, quantized = pl.pallas_call(
        _vq_block,
        grid=grid,
        in_specs=in_specs,
        out_specs=out_specs,
        out_shape=out_shape,
    )(o, a, *weights)
    return (re_p, re_o, latent, vq_latent, quantized)


# speedup vs baseline: 1.7801x; 1.7801x over previous
import numpy as np
import jax
import jax.numpy as jnp
from jax import lax
from jax.experimental import pallas as pl

B = 16384
HN = 8
OBS = HN + 1
ANUM = 2 ** HN
VQ_DIM = 4
VQ_SIZE = 8
LAT = 16
NN0, NN1 = 258, 128

BB = 2048
NG = LAT // VQ_DIM
W = VQ_SIZE * LAT

_eye128 = np.eye(128, dtype=np.float32)


def _rot(k):
    return np.eye(128, k=-k, dtype=np.float32)


_R64 = _rot(64)
_R32 = _rot(32)
_R16 = _rot(16)
_T16 = np.tile(np.eye(LAT, dtype=np.float32), (1, VQ_SIZE))
_LPR = np.kron(np.triu(np.ones((VQ_SIZE, VQ_SIZE), np.float32), 1),
               np.eye(LAT, dtype=np.float32))
_GRP = (np.arange(LAT)[:, None] // VQ_DIM ==
        np.arange(LAT)[None, :] // VQ_DIM).astype(np.float32)


def _vq_block(o_ref, a_ref, w1o_ref, w1a_ref, b1_ref, w2_ref, b2_ref,
              wl_ref, bl_ref, md_ref, cb2_ref, lpr_ref, q_ref_w, a1_ref, ab1_ref,
              a2_ref, ab2_ref, ap_ref, abp_ref,
              o2_ref, ob2_ref, oo_ref, obo_ref,
              rep_ref, reo_ref, lat_ref, vq_ref, quant_ref):
    f32 = jnp.float32
    xo = o_ref[...]
    a = a_ref[...]

    iota_a = lax.broadcasted_iota(jnp.int32, (BB, ANUM), 1)
    a_hot = jnp.where(iota_a == a, 1.0, 0.0).astype(f32)

    h = xo @ w1o_ref[...] + a_hot @ w1a_ref[...] + b1_ref[...]
    h = jnp.maximum(h, 0.0)
    h = jnp.maximum(h @ w2_ref[...] + b2_ref[...], 0.0)
    lat = h @ wl_ref[...] + bl_ref[...]
    lat_ref[...] = lat

    score = lat @ md_ref[...] + cb2_ref[...]
    m = jnp.minimum(score[:, :64], score[:, 64:])
    m = jnp.minimum(m[:, :32], m[:, 32:])
    m = jnp.minimum(m[:, :16], m[:, 16:])
    min_t = jnp.tile(m, (1, VQ_SIZE))
    onehot = jnp.where(score == min_t, 1.0, 0.0)
    cnt = onehot @ lpr_ref[...]
    first = jnp.where(cnt == 0.0, onehot, 0.0)
    q = first @ q_ref_w[...]
    vq = lat + (q - lat)
    vq_ref[...] = vq
    quant_ref[...] = q

    xa = jnp.concatenate([vq, xo], axis=1)
    hh = jnp.maximum(xa @ a1_ref[...] + ab1_ref[...], 0.0)
    ha = hh[:, :NN1]
    ho = hh[:, NN1:]
    ha = jnp.maximum(ha @ a2_ref[...] + ab2_ref[...], 0.0)
    rep_ref[...] = ha @ ap_ref[...] + abp_ref[...]
    ho = jnp.maximum(ho @ o2_ref[...] + ob2_ref[...], 0.0)
    reo_ref[...] = ho @ oo_ref[...] + obo_ref[...]


def kernel(o, a, enc_w1, enc_b1, enc_w2, enc_b2, enc_wl, enc_bl, codebook,
           ad_w1, ad_b1, ad_w2, ad_b2, ad_wp, ad_bp,
           od_w1, od_b1, od_w2, od_b2, od_wo, od_bo):
    f32 = jnp.float32
    w1o = enc_w1[:, :OBS].T
    w1a = enc_w1[:, OBS:].T
    w2 = enc_w2.T
    wl = enc_wl.T
    a2 = ad_w2.T
    ap = ad_wp.T
    o2 = od_w2.T
    oo = od_wo.T

    a1v = ad_w1[:, :LAT].T
    a1o = ad_w1[:, LAT:].T
    o1 = od_w1.T
    a1 = jnp.concatenate([
        jnp.concatenate([a1v, o1], axis=1),
        jnp.concatenate([a1o, jnp.zeros((OBS, NN1), f32)], axis=1),
    ], axis=0)
    ab1 = jnp.concatenate([ad_b1, od_b1]).reshape(1, -1)

    grp = jnp.asarray(_GRP)
    base = codebook[:, jnp.arange(LAT) % VQ_DIM]
    md = (-2.0 * base[:, :, None] * grp[None, :, :])
    md = jnp.transpose(md, (1, 0, 2)).reshape(LAT, W)
    cb2 = jnp.repeat(jnp.sum(codebook * codebook, axis=1), LAT).reshape(1, W)
    qm = (base[:, None, :] * grp[None, :, :] / 4.0)
    qm = qm.reshape(W, LAT)

    def b2d(v):
        return v.reshape(1, -1)

    grid = (B // BB,)

    def bcast_spec(arr):
        return pl.BlockSpec(arr.shape, lambda i: (0, 0))

    weights = [w1o, w1a, b2d(enc_b1), w2, b2d(enc_b2), wl, b2d(enc_bl),
               md, cb2, jnp.asarray(_LPR), qm,
               a1, ab1, a2, b2d(ad_b2), ap, b2d(ad_bp),
               o2, b2d(od_b2), oo, b2d(od_bo)]

    in_specs = [
        pl.BlockSpec((BB, OBS), lambda i: (i, 0)),
        pl.BlockSpec((BB, 1), lambda i: (i, 0)),
    ] + [bcast_spec(w) for w in weights]

    out_specs = (
        pl.BlockSpec((BB, ANUM), lambda i: (i, 0)),
        pl.BlockSpec((BB, OBS), lambda i: (i, 0)),
        pl.BlockSpec((BB, LAT), lambda i: (i, 0)),
        pl.BlockSpec((BB, LAT), lambda i: (i, 0)),
        pl.BlockSpec((BB, LAT), lambda i: (i, 0)),
    )
    out_shape = (
        jax.ShapeDtypeStruct((B, ANUM), f32),
        jax.ShapeDtypeStruct((B, OBS), f32),
        jax.ShapeDtypeStruct((B, LAT), f32),
        jax.ShapeDtypeStruct((B, LAT), f32),
        jax.ShapeDtypeStruct((B, LAT), f32),
    )

    re_p, re_o, latent, vq_latent, quantized = pl.pallas_call(
        _vq_block,
        grid=grid,
        in_specs=in_specs,
        out_specs=out_specs,
        out_shape=out_shape,
    )(o, a, *weights)
    return (re_p, re_o, latent, vq_latent, quantized)


# speedup vs baseline: 1.9097x; 1.0728x over previous
import numpy as np
import jax
import jax.numpy as jnp
from jax import lax
from jax.experimental import pallas as pl
from jax.experimental.pallas import tpu as pltpu

B = 16384
HN = 8
OBS = HN + 1
ANUM = 2 ** HN
VQ_DIM = 4
VQ_SIZE = 8
LAT = 16
NN0, NN1 = 258, 128

BB = 2048
NG = LAT // VQ_DIM
W = VQ_SIZE * LAT

_eye128 = np.eye(128, dtype=np.float32)


def _rot(k):
    return np.eye(128, k=-k, dtype=np.float32)


_R64 = _rot(64)
_R32 = _rot(32)
_R16 = _rot(16)
_T16 = np.tile(np.eye(LAT, dtype=np.float32), (1, VQ_SIZE))
_LPR = np.kron(np.triu(np.ones((VQ_SIZE, VQ_SIZE), np.float32), 1),
               np.eye(LAT, dtype=np.float32))
_GRP = (np.arange(LAT)[:, None] // VQ_DIM ==
        np.arange(LAT)[None, :] // VQ_DIM).astype(np.float32)


def _vq_block(o_ref, a_ref, w1o_ref, w1a_ref, b1_ref, w2_ref, b2_ref,
              wl_ref, bl_ref, md_ref, cb2_ref, lpr_ref, q_ref_w,
              a1_ref, a1o_ref, ab1_ref,
              a2_ref, ab2_ref, ap_ref, abp_ref,
              o2_ref, ob2_ref, oo_ref, obo_ref,
              rep_ref, reo_ref, lat_ref, vq_ref, quant_ref):
    f32 = jnp.float32
    xo = o_ref[...]
    a = a_ref[...]

    iota_a = lax.broadcasted_iota(jnp.int32, (BB, ANUM), 1)
    a_hot = jnp.where(iota_a == a, 1.0, 0.0).astype(f32)

    h = xo @ w1o_ref[...] + a_hot @ w1a_ref[...] + b1_ref[...]
    h = jnp.maximum(h, 0.0)
    h = jnp.maximum(h @ w2_ref[...] + b2_ref[...], 0.0)
    lat = h @ wl_ref[...] + bl_ref[...]
    lat_ref[...] = lat

    score = lat @ md_ref[...] + cb2_ref[...]
    s = jnp.minimum(score, pltpu.roll(score, 64, 1))
    s = jnp.minimum(s, pltpu.roll(s, 32, 1))
    min_t = jnp.minimum(s, pltpu.roll(s, 16, 1))
    onehot = jnp.where(score == min_t, 1.0, 0.0)
    cnt = onehot @ lpr_ref[...]
    first = jnp.where(cnt == 0.0, onehot, 0.0)
    q = first @ q_ref_w[...]
    vq = lat + (q - lat)
    vq_ref[...] = vq
    quant_ref[...] = q

    bf16 = jnp.bfloat16
    vqb = vq.astype(bf16)
    xob = xo.astype(bf16)

    def mm(x, w):
        return jnp.dot(x, w, preferred_element_type=jnp.float32)

    hh = jnp.maximum(mm(vqb, a1_ref[...]) + mm(xob, a1o_ref[...])
                     + ab1_ref[...], 0.0)
    ha = hh[:, :NN1].astype(bf16)
    ho = hh[:, NN1:].astype(bf16)
    ha = jnp.maximum(mm(ha, a2_ref[...]) + ab2_ref[...], 0.0).astype(bf16)
    rep_ref[...] = mm(ha, ap_ref[...]) + abp_ref[...]
    ho = jnp.maximum(mm(ho, o2_ref[...]) + ob2_ref[...], 0.0).astype(bf16)
    reo_ref[...] = mm(ho, oo_ref[...]) + obo_ref[...]


def kernel(o, a, enc_w1, enc_b1, enc_w2, enc_b2, enc_wl, enc_bl, codebook,
           ad_w1, ad_b1, ad_w2, ad_b2, ad_wp, ad_bp,
           od_w1, od_b1, od_w2, od_b2, od_wo, od_bo):
    f32 = jnp.float32
    w1o = enc_w1[:, :OBS].T
    w1a = enc_w1[:, OBS:].T
    w2 = enc_w2.T
    wl = enc_wl.T
    bf16 = jnp.bfloat16
    a2 = ad_w2.T.astype(bf16)
    ap = ad_wp.T.astype(bf16)
    o2 = od_w2.T.astype(bf16)
    oo = od_wo.T.astype(bf16)

    a1v = ad_w1[:, :LAT].T
    a1o_ = ad_w1[:, LAT:].T
    o1 = od_w1.T
    a1 = jnp.concatenate([a1v, o1], axis=1).astype(bf16)
    a1o = jnp.concatenate([a1o_, jnp.zeros((OBS, NN1), f32)],
                          axis=1).astype(bf16)
    ab1 = jnp.concatenate([ad_b1, od_b1]).reshape(1, -1)

    grp = jnp.asarray(_GRP)
    base = codebook[:, jnp.arange(LAT) % VQ_DIM]
    md = (-2.0 * base[:, :, None] * grp[None, :, :])
    md = jnp.transpose(md, (1, 0, 2)).reshape(LAT, W)
    cb2 = jnp.repeat(jnp.sum(codebook * codebook, axis=1), LAT).reshape(1, W)
    qm = (base[:, None, :] * grp[None, :, :] / 4.0)
    qm = qm.reshape(W, LAT)

    def b2d(v):
        return v.reshape(1, -1)

    grid = (B // BB,)

    def bcast_spec(arr):
        return pl.BlockSpec(arr.shape, lambda i: (0, 0))

    weights = [w1o, w1a, b2d(enc_b1), w2, b2d(enc_b2), wl, b2d(enc_bl),
               md, cb2, jnp.asarray(_LPR), qm,
               a1, a1o, ab1, a2, b2d(ad_b2), ap, b2d(ad_bp),
               o2, b2d(od_b2), oo, b2d(od_bo)]

    in_specs = [
        pl.BlockSpec((BB, OBS), lambda i: (i, 0)),
        pl.BlockSpec((BB, 1), lambda i: (i, 0)),
    ] + [bcast_spec(w) for w in weights]

    out_specs = (
        pl.BlockSpec((BB, ANUM), lambda i: (i, 0)),
        pl.BlockSpec((BB, OBS), lambda i: (i, 0)),
        pl.BlockSpec((BB, LAT), lambda i: (i, 0)),
        pl.BlockSpec((BB, LAT), lambda i: (i, 0)),
        pl.BlockSpec((BB, LAT), lambda i: (i, 0)),
    )
    out_shape = (
        jax.ShapeDtypeStruct((B, ANUM), f32),
        jax.ShapeDtypeStruct((B, OBS), f32),
        jax.ShapeDtypeStruct((B, LAT), f32),
        jax.ShapeDtypeStruct((B, LAT), f32),
        jax.ShapeDtypeStruct((B, LAT), f32),
    )

    re_p, re_o, latent, vq_latent, quantized = pl.pallas_call(
        _vq_block,
        grid=grid,
        in_specs=in_specs,
        out_specs=out_specs,
        out_shape=out_shape,
    )(o, a, *weights)
    return (re_p, re_o, latent, vq_latent, quantized)


# speedup vs baseline: 2.1867x; 1.1451x over previous
import numpy as np
import jax
import jax.numpy as jnp
from jax import lax
from jax.experimental import pallas as pl
from jax.experimental.pallas import tpu as pltpu

B = 16384
HN = 8
OBS = HN + 1
ANUM = 2 ** HN
VQ_DIM = 4
VQ_SIZE = 8
LAT = 16
NN0, NN1 = 258, 128

BB = 2048
W = VQ_SIZE * LAT

_S4 = np.tile(np.eye(VQ_DIM, dtype=np.float32), (1, VQ_DIM))
_E8 = np.kron(np.eye(VQ_SIZE, dtype=np.float32),
              np.ones((1, LAT), np.float32))
_GRP = (np.arange(LAT)[:, None] // VQ_DIM ==
        np.arange(LAT)[None, :] // VQ_DIM).astype(np.float32)
_G1 = np.tile(_GRP, (1, VQ_SIZE))
_G2 = np.tile(_GRP, (VQ_SIZE, 1))
_LPR = np.kron(np.triu(np.ones((VQ_SIZE, VQ_SIZE), np.float32), 1),
               np.eye(LAT, dtype=np.float32))

_DN_T = (((1,), (1,)), ((), ()))


def _dt(x, w):
    return lax.dot_general(x, w, _DN_T, preferred_element_type=jnp.float32)


def _d0(x, w):
    return lax.dot_general(x, w, (((0,), (0,)), ((), ())),
                           preferred_element_type=jnp.float32)


def _vq_block(o_ref, a_ref, w1_ref, b1_ref, w2_ref, b2_ref, wl_ref, bl_ref,
              cb_ref, aw1_ref, ab1_ref, aw2_ref, ab2_ref, awp_ref, abp_ref,
              ow1_ref, ob1_ref, ow2_ref, ob2_ref, owo_ref, obo_ref,
              s4_ref, e8_ref, g1_ref, g2_ref, lpr_ref,
              rep_ref, reo_ref, lat_ref, vq_ref, quant_ref):
    f32 = jnp.float32
    bf16 = jnp.bfloat16
    xo = o_ref[...]
    a = a_ref[...]

    iota_a = lax.broadcasted_iota(jnp.int32, (BB, ANUM), 1)
    a_hot = jnp.where(iota_a == a, 1.0, 0.0).astype(f32)

    w1 = w1_ref[...]
    h = _dt(xo, w1[:, :OBS]) + _dt(a_hot, w1[:, OBS:]) + b1_ref[...]
    h = jnp.maximum(h, 0.0)
    h = jnp.maximum(_dt(h, w2_ref[...]) + b2_ref[...], 0.0)
    lat = _dt(h, wl_ref[...]) + bl_ref[...]
    lat_ref[...] = lat

    cb = cb_ref[...]
    base16 = jnp.dot(cb, s4_ref[...], preferred_element_type=f32)
    md = _d0(-2.0 * base16, e8_ref[...]) * g1_ref[...]
    cb2v = _d0(jnp.sum(cb * cb, axis=1, keepdims=True), e8_ref[...])
    qm = _d0(e8_ref[...], base16 * 0.25) * g2_ref[...]

    score = jnp.dot(lat, md, preferred_element_type=f32) + cb2v
    s = jnp.minimum(score, pltpu.roll(score, 64, 1))
    s = jnp.minimum(s, pltpu.roll(s, 32, 1))
    min_t = jnp.minimum(s, pltpu.roll(s, 16, 1))
    onehot = jnp.where(score == min_t, 1.0, 0.0)
    cnt = jnp.dot(onehot, lpr_ref[...], preferred_element_type=f32)
    first = jnp.where(cnt == 0.0, onehot, 0.0)
    q = jnp.dot(first, qm, preferred_element_type=f32)
    vq = lat + (q - lat)
    vq_ref[...] = vq
    quant_ref[...] = q

    vqb = vq.astype(bf16)
    xob = xo.astype(bf16)
    aw1 = aw1_ref[...]
    ha = _dt(vqb, aw1[:, :LAT].astype(bf16)) + \
        _dt(xob, aw1[:, LAT:].astype(bf16)) + ab1_ref[...]
    ha = jnp.maximum(ha, 0.0).astype(bf16)
    ha = jnp.maximum(_dt(ha, aw2_ref[...].astype(bf16)) + ab2_ref[...],
                     0.0).astype(bf16)
    rep_ref[...] = _dt(ha, awp_ref[...].astype(bf16)) + abp_ref[...]

    ho = jnp.maximum(_dt(vqb, ow1_ref[...].astype(bf16)) + ob1_ref[...],
                     0.0).astype(bf16)
    ho = jnp.maximum(_dt(ho, ow2_ref[...].astype(bf16)) + ob2_ref[...],
                     0.0).astype(bf16)
    reo_ref[...] = _dt(ho, owo_ref[...].astype(bf16)) + obo_ref[...]


def kernel(o, a, enc_w1, enc_b1, enc_w2, enc_b2, enc_wl, enc_bl, codebook,
           ad_w1, ad_b1, ad_w2, ad_b2, ad_wp, ad_bp,
           od_w1, od_b1, od_w2, od_b2, od_wo, od_bo):
    f32 = jnp.float32

    def b2d(v):
        return v.reshape(1, -1)

    consts = [jnp.asarray(_S4), jnp.asarray(_E8), jnp.asarray(_G1),
              jnp.asarray(_G2), jnp.asarray(_LPR)]
    weights = [enc_w1, b2d(enc_b1), enc_w2, b2d(enc_b2), enc_wl, b2d(enc_bl),
               codebook, ad_w1, b2d(ad_b1), ad_w2, b2d(ad_b2), ad_wp,
               b2d(ad_bp), od_w1, b2d(od_b1), od_w2, b2d(od_b2), od_wo,
               b2d(od_bo)] + consts

    def bcast_spec(arr):
        return pl.BlockSpec(arr.shape, lambda i: (0, 0))

    in_specs = [
        pl.BlockSpec((BB, OBS), lambda i: (i, 0)),
        pl.BlockSpec((BB, 1), lambda i: (i, 0)),
    ] + [bcast_spec(w) for w in weights]

    out_specs = (
        pl.BlockSpec((BB, ANUM), lambda i: (i, 0)),
        pl.BlockSpec((BB, OBS), lambda i: (i, 0)),
        pl.BlockSpec((BB, LAT), lambda i: (i, 0)),
        pl.BlockSpec((BB, LAT), lambda i: (i, 0)),
        pl.BlockSpec((BB, LAT), lambda i: (i, 0)),
    )
    out_shape = (
        jax.ShapeDtypeStruct((B, ANUM), f32),
        jax.ShapeDtypeStruct((B, OBS), f32),
        jax.ShapeDtypeStruct((B, LAT), f32),
        jax.ShapeDtypeStruct((B, LAT), f32),
        jax.ShapeDtypeStruct((B, LAT), f32),
    )

    re_p, re_o, latent, vq_latent, quantized = pl.pallas_call(
        _vq_block,
        grid=(B // BB,),
        in_specs=in_specs,
        out_specs=out_specs,
        out_shape=out_shape,
    )(o, a, *weights)
    return (re_p, re_o, latent, vq_latent, quantized)


# speedup vs baseline: 2.2208x; 1.0156x over previous
import numpy as np
import jax
import jax.numpy as jnp
from jax import lax
from jax.experimental import pallas as pl
from jax.experimental.pallas import tpu as pltpu

B = 16384
HN = 8
OBS = HN + 1
ANUM = 2 ** HN
VQ_DIM = 4
VQ_SIZE = 8
LAT = 16
NN0, NN1 = 258, 128

BB = 2048
W = VQ_SIZE * LAT

_S4 = np.tile(np.eye(VQ_DIM, dtype=np.float32), (1, VQ_DIM))
_E8 = np.kron(np.eye(VQ_SIZE, dtype=np.float32),
              np.ones((1, LAT), np.float32))
_GRP = (np.arange(LAT)[:, None] // VQ_DIM ==
        np.arange(LAT)[None, :] // VQ_DIM).astype(np.float32)
_G1 = np.tile(_GRP, (1, VQ_SIZE))
_G2 = np.tile(_GRP, (VQ_SIZE, 1))
_LPR = np.kron(np.triu(np.ones((VQ_SIZE, VQ_SIZE), np.float32), 1),
               np.eye(LAT, dtype=np.float32))

_DN_T = (((1,), (1,)), ((), ()))


def _dt(x, w):
    return lax.dot_general(x, w, _DN_T, preferred_element_type=jnp.float32)


def _vq_block(o_ref, a_ref, w1_ref, b1_ref, w2_ref, b2_ref, wl_ref, bl_ref,
              cb_ref, aw1_ref, ab1_ref, aw2_ref, ab2_ref, awp_ref, abp_ref,
              ow1_ref, ob1_ref, ow2_ref, ob2_ref, owo_ref, obo_ref,
              g2_ref, lpr_ref,
              rep_ref, reo_ref, lat_ref, vq_ref, quant_ref):
    f32 = jnp.float32
    bf16 = jnp.bfloat16
    xo = o_ref[...]
    a = a_ref[...]

    iota_a = lax.broadcasted_iota(jnp.int32, (BB, ANUM), 1)
    a_hot = jnp.where(iota_a == a, 1.0, 0.0).astype(f32)

    w1 = w1_ref[...]
    h = _dt(xo, w1[:, :OBS]) + _dt(a_hot, w1[:, OBS:]) + b1_ref[...]
    h = jnp.maximum(h, 0.0)
    h = jnp.maximum(_dt(h, w2_ref[...]) + b2_ref[...], 0.0)
    lat = _dt(h, wl_ref[...]) + bl_ref[...]
    lat_ref[...] = lat

    cb = cb_ref[...]
    cbpat = jnp.tile(jnp.repeat(cb, LAT, axis=0), (1, VQ_DIM)) * g2_ref[...]
    mdT = -2.0 * cbpat
    qm = 0.25 * cbpat
    cb2v = jnp.transpose(
        jnp.sum(cbpat * cbpat, axis=1, keepdims=True))

    score = _dt(lat, mdT) + cb2v
    s = jnp.minimum(score, pltpu.roll(score, 64, 1))
    s = jnp.minimum(s, pltpu.roll(s, 32, 1))
    min_t = jnp.minimum(s, pltpu.roll(s, 16, 1))
    onehot = jnp.where(score == min_t, 1.0, 0.0)
    cnt = jnp.dot(onehot, lpr_ref[...], preferred_element_type=f32)
    first = jnp.where(cnt == 0.0, onehot, 0.0)
    q = jnp.dot(first, qm, preferred_element_type=f32)
    vq = lat + (q - lat)
    vq_ref[...] = vq
    quant_ref[...] = q

    vqb = vq.astype(bf16)
    xob = xo.astype(bf16)
    aw1 = aw1_ref[...]
    ha = _dt(vqb, aw1[:, :LAT].astype(bf16)) + \
        _dt(xob, aw1[:, LAT:].astype(bf16)) + ab1_ref[...]
    ha = jnp.maximum(ha, 0.0).astype(bf16)
    ha = jnp.maximum(_dt(ha, aw2_ref[...].astype(bf16)) + ab2_ref[...],
                     0.0).astype(bf16)
    rep_ref[...] = _dt(ha, awp_ref[...].astype(bf16)) + abp_ref[...]

    ho = jnp.maximum(_dt(vqb, ow1_ref[...].astype(bf16)) + ob1_ref[...],
                     0.0).astype(bf16)
    ho = jnp.maximum(_dt(ho, ow2_ref[...].astype(bf16)) + ob2_ref[...],
                     0.0).astype(bf16)
    reo_ref[...] = _dt(ho, owo_ref[...].astype(bf16)) + obo_ref[...]


def kernel(o, a, enc_w1, enc_b1, enc_w2, enc_b2, enc_wl, enc_bl, codebook,
           ad_w1, ad_b1, ad_w2, ad_b2, ad_wp, ad_bp,
           od_w1, od_b1, od_w2, od_b2, od_wo, od_bo):
    f32 = jnp.float32

    def b2d(v):
        return v.reshape(1, -1)

    consts = [jnp.asarray(_G2), jnp.asarray(_LPR)]
    weights = [enc_w1, b2d(enc_b1), enc_w2, b2d(enc_b2), enc_wl, b2d(enc_bl),
               codebook, ad_w1, b2d(ad_b1), ad_w2, b2d(ad_b2), ad_wp,
               b2d(ad_bp), od_w1, b2d(od_b1), od_w2, b2d(od_b2), od_wo,
               b2d(od_bo)] + consts

    def bcast_spec(arr):
        return pl.BlockSpec(arr.shape, lambda i: (0, 0))

    in_specs = [
        pl.BlockSpec((BB, OBS), lambda i: (i, 0)),
        pl.BlockSpec((BB, 1), lambda i: (i, 0)),
    ] + [bcast_spec(w) for w in weights]

    out_specs = (
        pl.BlockSpec((BB, ANUM), lambda i: (i, 0)),
        pl.BlockSpec((BB, OBS), lambda i: (i, 0)),
        pl.BlockSpec((BB, LAT), lambda i: (i, 0)),
        pl.BlockSpec((BB, LAT), lambda i: (i, 0)),
        pl.BlockSpec((BB, LAT), lambda i: (i, 0)),
    )
    out_shape = (
        jax.ShapeDtypeStruct((B, ANUM), f32),
        jax.ShapeDtypeStruct((B, OBS), f32),
        jax.ShapeDtypeStruct((B, LAT), f32),
        jax.ShapeDtypeStruct((B, LAT), f32),
        jax.ShapeDtypeStruct((B, LAT), f32),
    )

    re_p, re_o, latent, vq_latent, quantized = pl.pallas_call(
        _vq_block,
        grid=(B // BB,),
        in_specs=in_specs,
        out_specs=out_specs,
        out_shape=out_shape,
    )(o, a, *weights)
    return (re_p, re_o, latent, vq_latent, quantized)


# speedup vs baseline: 2.7108x; 1.2206x over previous
import numpy as np
import jax
import jax.numpy as jnp
from jax import lax
from jax.experimental import pallas as pl
from jax.experimental.pallas import tpu as pltpu

B = 16384
HN = 8
OBS = HN + 1
ANUM = 2 ** HN
VQ_DIM = 4
VQ_SIZE = 8
LAT = 16
NN0, NN1 = 258, 128

BB = 2048
W = VQ_SIZE * LAT

_GRP = (np.arange(LAT)[:, None] // VQ_DIM ==
        np.arange(LAT)[None, :] // VQ_DIM).astype(np.float32)
_G2 = np.tile(_GRP, (VQ_SIZE, 1))
_LPR = np.kron(np.triu(np.ones((VQ_SIZE, VQ_SIZE), np.float32), 1),
               np.eye(LAT, dtype=np.float32))


def _dt(x, w):
    return lax.dot_general(x, w, (((1,), (1,)), ((), ())),
                           preferred_element_type=jnp.float32)


def _dx(xt, w):
    return lax.dot_general(xt, w, (((0,), (0,)), ((), ())),
                           preferred_element_type=jnp.float32)


def _vq_block(ot_ref, a_ref, w1_ref, b1_ref, w2t_ref, b2_ref, wl_ref, bl_ref,
              cbt_ref, aw1t_ref, ab1_ref, aw2_ref, ab2_ref, awpt_ref, abp_ref,
              ow1t_ref, ob1_ref, ow2_ref, ob2_ref, owo_ref, obo_ref,
              g2_ref, lpr_ref,
              rep_ref, reo_ref, lat_ref, vq_ref, quant_ref):
    f32 = jnp.float32
    bf16 = jnp.bfloat16
    xot = ot_ref[...]
    a = a_ref[...]

    iota_a = lax.broadcasted_iota(jnp.int32, (BB, ANUM), 1)
    a_hot = jnp.where(iota_a == a, 1.0, 0.0).astype(f32)

    w1 = w1_ref[...]
    h = _dx(xot, jnp.transpose(w1[:, :OBS])) \
        + _dt(a_hot, w1[:, OBS:]) + b1_ref[...][None, :]
    h = jnp.maximum(h, 0.0)
    h = jnp.maximum(jnp.dot(h, w2t_ref[...], preferred_element_type=f32)
                    + b2_ref[...][None, :], 0.0)
    lat = _dt(h, wl_ref[...]) + bl_ref[...][None, :]
    lat_ref[...] = lat

    cb = jnp.transpose(cbt_ref[...])
    cbpat = jnp.tile(jnp.repeat(cb, LAT, axis=0), (1, VQ_DIM)) * g2_ref[...]
    mdT = -2.0 * cbpat
    qm = 0.25 * cbpat
    cb2v = jnp.transpose(
        jnp.sum(cbpat * cbpat, axis=1, keepdims=True))

    score = _dt(lat, mdT) + cb2v
    s = jnp.minimum(score, pltpu.roll(score, 64, 1))
    s = jnp.minimum(s, pltpu.roll(s, 32, 1))
    min_t = jnp.minimum(s, pltpu.roll(s, 16, 1))
    onehot = jnp.where(score == min_t, 1.0, 0.0)
    cnt = jnp.dot(onehot, lpr_ref[...], preferred_element_type=f32)
    first = jnp.where(cnt == 0.0, onehot, 0.0)
    q = jnp.dot(first, qm, preferred_element_type=f32)
    vq = lat + (q - lat)
    vq_ref[...] = vq
    quant_ref[...] = q

    vqb = vq.astype(bf16)
    xotb = xot.astype(bf16)
    aw1t = aw1t_ref[...]
    ha = jnp.dot(vqb, aw1t[:LAT].astype(bf16), preferred_element_type=f32) \
        + _dx(xotb, aw1t[LAT:].astype(bf16)) + ab1_ref[...][None, :]
    ha = jnp.maximum(ha, 0.0).astype(bf16)
    ha = jnp.maximum(_dt(ha, aw2_ref[...].astype(bf16))
                     + ab2_ref[...][None, :], 0.0).astype(bf16)
    rep_ref[...] = jnp.dot(ha, awpt_ref[...].astype(bf16),
                           preferred_element_type=f32) + abp_ref[...][None, :]

    ho = jnp.maximum(jnp.dot(vqb, ow1t_ref[...].astype(bf16),
                             preferred_element_type=f32)
                     + ob1_ref[...][None, :], 0.0).astype(bf16)
    ho = jnp.maximum(_dt(ho, ow2_ref[...].astype(bf16))
                     + ob2_ref[...][None, :], 0.0).astype(bf16)
    reo_ref[...] = _dt(ho, owo_ref[...].astype(bf16)) \
        + obo_ref[...][None, :]


def kernel(o, a, enc_w1, enc_b1, enc_w2, enc_b2, enc_wl, enc_bl, codebook,
           ad_w1, ad_b1, ad_w2, ad_b2, ad_wp, ad_bp,
           od_w1, od_b1, od_w2, od_b2, od_wo, od_bo):
    f32 = jnp.float32

    consts = [jnp.asarray(_G2), jnp.asarray(_LPR)]
    weights = [enc_w1, enc_b1, enc_w2.T, enc_b2, enc_wl, enc_bl,
               codebook.T, ad_w1.T, ad_b1, ad_w2, ad_b2, ad_wp.T,
               ad_bp, od_w1.T, od_b1, od_w2, od_b2, od_wo,
               od_bo] + consts

    def bcast_spec(arr):
        if arr.ndim == 1:
            return pl.BlockSpec(arr.shape, lambda i: (0,))
        return pl.BlockSpec(arr.shape, lambda i: (0, 0))

    in_specs = [
        pl.BlockSpec((OBS, BB), lambda i: (0, i)),
        pl.BlockSpec((BB, 1), lambda i: (i, 0)),
    ] + [bcast_spec(w) for w in weights]

    out_specs = (
        pl.BlockSpec((BB, ANUM), lambda i: (i, 0)),
        pl.BlockSpec((BB, OBS), lambda i: (i, 0)),
        pl.BlockSpec((BB, LAT), lambda i: (i, 0)),
        pl.BlockSpec((BB, LAT), lambda i: (i, 0)),
        pl.BlockSpec((BB, LAT), lambda i: (i, 0)),
    )
    out_shape = (
        jax.ShapeDtypeStruct((B, ANUM), f32),
        jax.ShapeDtypeStruct((B, OBS), f32),
        jax.ShapeDtypeStruct((B, LAT), f32),
        jax.ShapeDtypeStruct((B, LAT), f32),
        jax.ShapeDtypeStruct((B, LAT), f32),
    )

    re_p, re_o, latent, vq_latent, quantized = pl.pallas_call(
        _vq_block,
        grid=(B // BB,),
        in_specs=in_specs,
        out_specs=out_specs,
        out_shape=out_shape,
    )(o.T, a, *weights)
    return (re_p, re_o, latent, vq_latent, quantized)


# speedup vs baseline: 4.8783x; 1.7996x over previous
import numpy as np
import jax
import jax.numpy as jnp
from jax import lax
from jax.experimental import pallas as pl
from jax.experimental.pallas import tpu as pltpu

B = 16384
HN = 8
OBS = HN + 1
ANUM = 2 ** HN
VQ_DIM = 4
VQ_SIZE = 8
LAT = 16
NN0, NN1 = 258, 128

BB = 2048
W = VQ_SIZE * LAT

_GRP = (np.arange(LAT)[:, None] // VQ_DIM ==
        np.arange(LAT)[None, :] // VQ_DIM).astype(np.float32)
_G1 = np.tile(_GRP, (1, VQ_SIZE))
_LPR = np.kron(np.triu(np.ones((VQ_SIZE, VQ_SIZE), np.float32), 1),
               np.eye(LAT, dtype=np.float32))


def _mm(x, w):
    return jnp.dot(x, w, preferred_element_type=jnp.float32)


def _dt(x, w):
    return lax.dot_general(x, w, (((1,), (1,)), ((), ())),
                           preferred_element_type=jnp.float32)


def _dx(xt, w):
    return lax.dot_general(xt, w, (((0,), (0,)), ((), ())),
                           preferred_element_type=jnp.float32)


def _vq_block(ot_ref, a_ref, w1_ref, b1_ref, w2t_ref, b2_ref, wl_ref, bl_ref,
              cbt_ref, aw1t_ref, ab1_ref, aw2_ref, ab2_ref, awpt_ref, abp_ref,
              ow1t_ref, ob1_ref, ow2_ref, ob2_ref, owo_ref, obo_ref,
              g1_ref, lpr_ref,
              rep_ref, reot_ref, latt_ref, vqt_ref, quantt_ref):
    f32 = jnp.float32
    bf16 = jnp.bfloat16
    xot = ot_ref[...]
    a = a_ref[...]

    iota_v = lax.broadcasted_iota(jnp.int32, (ANUM, BB), 0)
    a_hott = jnp.where(iota_v == a, 1.0, 0.0).astype(f32)

    w1 = w1_ref[...]
    ht = _mm(w1[:, OBS:], a_hott) + _mm(w1[:, :OBS], xot) \
        + b1_ref[...][:, None]
    ht = jnp.maximum(ht, 0.0)
    h2t = jnp.maximum(_dx(w2t_ref[...], ht) + b2_ref[...][:, None],
                      0.0)
    latt = _mm(wl_ref[...], h2t) + bl_ref[...][:, None]
    latt_ref[...] = latt

    cbt = cbt_ref[...]
    md = -2.0 * jnp.repeat(jnp.tile(cbt, (VQ_DIM, 1)), LAT, axis=1) \
        * g1_ref[...]
    qmt = -0.125 * md
    cb2v = jnp.sum(0.25 * md * md, axis=0, keepdims=True)

    score = _dx(latt, md) + cb2v
    s = jnp.minimum(score, pltpu.roll(score, 64, 1))
    s = jnp.minimum(s, pltpu.roll(s, 32, 1))
    min_t = jnp.minimum(s, pltpu.roll(s, 16, 1))
    onehot = jnp.where(score == min_t, 1.0, 0.0)
    cnt = _mm(onehot, lpr_ref[...])
    first = jnp.where(cnt == 0.0, onehot, 0.0)
    qt = _dt(qmt, first)
    vqt = latt + (qt - latt)
    vqt_ref[...] = vqt
    quantt_ref[...] = qt

    vqbt = vqt.astype(bf16)
    xotb = xot.astype(bf16)
    aw1t = aw1t_ref[...]
    ha = _dx(vqbt, aw1t[:LAT].astype(bf16)) \
        + _dx(xotb, aw1t[LAT:].astype(bf16)) + ab1_ref[...][None, :]
    ha = jnp.maximum(ha, 0.0).astype(bf16)
    ha = jnp.maximum(_dt(ha, aw2_ref[...].astype(bf16))
                     + ab2_ref[...][None, :], 0.0).astype(bf16)
    rep_ref[...] = _mm(ha, awpt_ref[...].astype(bf16)) \
        + abp_ref[...][None, :]

    ho = jnp.maximum(_dx(vqbt, ow1t_ref[...].astype(bf16))
                     + ob1_ref[...][None, :], 0.0).astype(bf16)
    ho = jnp.maximum(_dt(ho, ow2_ref[...].astype(bf16))
                     + ob2_ref[...][None, :], 0.0).astype(bf16)
    reot_ref[...] = _dt(owo_ref[...].astype(bf16), ho) \
        + obo_ref[...][:, None]


def kernel(o, a, enc_w1, enc_b1, enc_w2, enc_b2, enc_wl, enc_bl, codebook,
           ad_w1, ad_b1, ad_w2, ad_b2, ad_wp, ad_bp,
           od_w1, od_b1, od_w2, od_b2, od_wo, od_bo):
    f32 = jnp.float32

    consts = [jnp.asarray(_G1), jnp.asarray(_LPR)]
    weights = [enc_w1, enc_b1, enc_w2.T, enc_b2, enc_wl, enc_bl,
               codebook.T, ad_w1.T, ad_b1, ad_w2, ad_b2, ad_wp.T,
               ad_bp, od_w1.T, od_b1, od_w2, od_b2, od_wo,
               od_bo] + consts

    def bcast_spec(arr):
        if arr.ndim == 1:
            return pl.BlockSpec(arr.shape, lambda i: (0,))
        return pl.BlockSpec(arr.shape, lambda i: (0, 0))

    in_specs = [
        pl.BlockSpec((OBS, BB), lambda i: (0, i)),
        pl.BlockSpec((1, BB), lambda i: (0, i)),
    ] + [bcast_spec(w) for w in weights]

    out_specs = (
        pl.BlockSpec((BB, ANUM), lambda i: (i, 0)),
        pl.BlockSpec((OBS, BB), lambda i: (0, i)),
        pl.BlockSpec((LAT, BB), lambda i: (0, i)),
        pl.BlockSpec((LAT, BB), lambda i: (0, i)),
        pl.BlockSpec((LAT, BB), lambda i: (0, i)),
    )
    out_shape = (
        jax.ShapeDtypeStruct((B, ANUM), f32),
        jax.ShapeDtypeStruct((OBS, B), f32),
        jax.ShapeDtypeStruct((LAT, B), f32),
        jax.ShapeDtypeStruct((LAT, B), f32),
        jax.ShapeDtypeStruct((LAT, B), f32),
    )

    re_p, re_ot, latentt, vq_latentt, quantizedt = pl.pallas_call(
        _vq_block,
        grid=(B // BB,),
        in_specs=in_specs,
        out_specs=out_specs,
        out_shape=out_shape,
    )(o.T, a.reshape(1, B), *weights)
    return (re_p, re_ot.T, latentt.T, vq_latentt.T, quantizedt.T)


# speedup vs baseline: 5.1259x; 1.0508x over previous
import numpy as np
import jax
import jax.numpy as jnp
from jax import lax
from jax.experimental import pallas as pl
from jax.experimental.pallas import tpu as pltpu

B = 16384
HN = 8
OBS = HN + 1
ANUM = 2 ** HN
VQ_DIM = 4
VQ_SIZE = 8
LAT = 16
NN0, NN1 = 258, 128

BB = 4096
W = VQ_SIZE * LAT

_GRP = (np.arange(LAT)[:, None] // VQ_DIM ==
        np.arange(LAT)[None, :] // VQ_DIM).astype(np.float32)
_G1 = np.tile(_GRP, (1, VQ_SIZE))
_LPR = np.kron(np.triu(np.ones((VQ_SIZE, VQ_SIZE), np.float32), 1),
               np.eye(LAT, dtype=np.float32))


def _mm(x, w):
    return jnp.dot(x, w, preferred_element_type=jnp.float32)


def _dt(x, w):
    return lax.dot_general(x, w, (((1,), (1,)), ((), ())),
                           preferred_element_type=jnp.float32)


def _dx(xt, w):
    return lax.dot_general(xt, w, (((0,), (0,)), ((), ())),
                           preferred_element_type=jnp.float32)


def _vq_block(ot_ref, a_ref, w1_ref, b1_ref, w2t_ref, b2_ref, wl_ref, bl_ref,
              cbt_ref, aw1t_ref, ab1_ref, aw2_ref, ab2_ref, awpt_ref, abp_ref,
              ow1t_ref, ob1_ref, ow2_ref, ob2_ref, owo_ref, obo_ref,
              g1_ref, lpr_ref,
              rep_ref, reot_ref, latt_ref, vqt_ref, quantt_ref):
    f32 = jnp.float32
    bf16 = jnp.bfloat16
    xot = ot_ref[...]
    a = a_ref[...]

    iota_v = lax.broadcasted_iota(jnp.int32, (ANUM, BB), 0)
    a_hott = jnp.where(iota_v == a, 1.0, 0.0).astype(f32)

    w1 = w1_ref[...]
    ht = _mm(w1[:, OBS:], a_hott) + _mm(w1[:, :OBS], xot) \
        + b1_ref[...][:, None]
    ht = jnp.maximum(ht, 0.0)
    h2t = jnp.maximum(_dx(w2t_ref[...], ht) + b2_ref[...][:, None],
                      0.0)
    latt = _mm(wl_ref[...], h2t) + bl_ref[...][:, None]
    latt_ref[...] = latt

    cbt = cbt_ref[...]
    md = -2.0 * jnp.repeat(jnp.tile(cbt, (VQ_DIM, 1)), LAT, axis=1) \
        * g1_ref[...]
    qmt = -0.125 * md
    cb2v = jnp.sum(0.25 * md * md, axis=0, keepdims=True)

    score = _dx(latt, md) + cb2v
    s = jnp.minimum(score, pltpu.roll(score, 64, 1))
    s = jnp.minimum(s, pltpu.roll(s, 32, 1))
    min_t = jnp.minimum(s, pltpu.roll(s, 16, 1))
    onehot = jnp.where(score == min_t, 1.0, 0.0)
    cnt = _mm(onehot, lpr_ref[...])
    first = jnp.where(cnt == 0.0, onehot, 0.0)
    qt = _dt(qmt, first)
    vqt = latt + (qt - latt)
    vqt_ref[...] = vqt
    quantt_ref[...] = qt

    vqbt = vqt.astype(bf16)
    xotb = xot.astype(bf16)
    aw1t = aw1t_ref[...]
    ha = _dx(vqbt, aw1t[:LAT].astype(bf16)) \
        + _dx(xotb, aw1t[LAT:].astype(bf16)) + ab1_ref[...][None, :]
    ha = jnp.maximum(ha, 0.0).astype(bf16)
    ha = jnp.maximum(_dt(ha, aw2_ref[...].astype(bf16))
                     + ab2_ref[...][None, :], 0.0).astype(bf16)
    rep_ref[...] = _mm(ha, awpt_ref[...].astype(bf16)) \
        + abp_ref[...][None, :]

    ho = jnp.maximum(_dx(vqbt, ow1t_ref[...].astype(bf16))
                     + ob1_ref[...][None, :], 0.0).astype(bf16)
    ho = jnp.maximum(_dt(ho, ow2_ref[...].astype(bf16))
                     + ob2_ref[...][None, :], 0.0).astype(bf16)
    reot_ref[...] = _dt(owo_ref[...].astype(bf16), ho) \
        + obo_ref[...][:, None]


def kernel(o, a, enc_w1, enc_b1, enc_w2, enc_b2, enc_wl, enc_bl, codebook,
           ad_w1, ad_b1, ad_w2, ad_b2, ad_wp, ad_bp,
           od_w1, od_b1, od_w2, od_b2, od_wo, od_bo):
    f32 = jnp.float32

    consts = [jnp.asarray(_G1), jnp.asarray(_LPR)]
    weights = [enc_w1, enc_b1, enc_w2.T, enc_b2, enc_wl, enc_bl,
               codebook.T, ad_w1.T, ad_b1, ad_w2, ad_b2, ad_wp.T,
               ad_bp, od_w1.T, od_b1, od_w2, od_b2, od_wo,
               od_bo] + consts

    def bcast_spec(arr):
        if arr.ndim == 1:
            return pl.BlockSpec(arr.shape, lambda i: (0,))
        return pl.BlockSpec(arr.shape, lambda i: (0, 0))

    in_specs = [
        pl.BlockSpec((OBS, BB), lambda i: (0, i)),
        pl.BlockSpec((1, BB), lambda i: (0, i)),
    ] + [bcast_spec(w) for w in weights]

    out_specs = (
        pl.BlockSpec((BB, ANUM), lambda i: (i, 0)),
        pl.BlockSpec((OBS, BB), lambda i: (0, i)),
        pl.BlockSpec((LAT, BB), lambda i: (0, i)),
        pl.BlockSpec((LAT, BB), lambda i: (0, i)),
        pl.BlockSpec((LAT, BB), lambda i: (0, i)),
    )
    out_shape = (
        jax.ShapeDtypeStruct((B, ANUM), f32),
        jax.ShapeDtypeStruct((OBS, B), f32),
        jax.ShapeDtypeStruct((LAT, B), f32),
        jax.ShapeDtypeStruct((LAT, B), f32),
        jax.ShapeDtypeStruct((LAT, B), f32),
    )

    re_p, re_ot, latentt, vq_latentt, quantizedt = pl.pallas_call(
        _vq_block,
        grid=(B // BB,),
        in_specs=in_specs,
        out_specs=out_specs,
        out_shape=out_shape,
    )(o.T, a.reshape(1, B), *weights)
    return (re_p, re_ot.T, latentt.T, vq_latentt.T, quantizedt.T)


# speedup vs baseline: 5.1280x; 1.0004x over previous
import numpy as np
import jax
import jax.numpy as jnp
from jax import lax
from jax.experimental import pallas as pl
from jax.experimental.pallas import tpu as pltpu

B = 16384
HN = 8
OBS = HN + 1
ANUM = 2 ** HN
VQ_DIM = 4
VQ_SIZE = 8
LAT = 16
NN0, NN1 = 258, 128

BB = 8192
W = VQ_SIZE * LAT

_GRP = (np.arange(LAT)[:, None] // VQ_DIM ==
        np.arange(LAT)[None, :] // VQ_DIM).astype(np.float32)
_G1 = np.tile(_GRP, (1, VQ_SIZE))
_LPR = np.kron(np.triu(np.ones((VQ_SIZE, VQ_SIZE), np.float32), 1),
               np.eye(LAT, dtype=np.float32))


def _mm(x, w):
    return jnp.dot(x, w, preferred_element_type=jnp.float32)


def _dt(x, w):
    return lax.dot_general(x, w, (((1,), (1,)), ((), ())),
                           preferred_element_type=jnp.float32)


def _dx(xt, w):
    return lax.dot_general(xt, w, (((0,), (0,)), ((), ())),
                           preferred_element_type=jnp.float32)


def _vq_block(ot_ref, a_ref, w1_ref, b1_ref, w2t_ref, b2_ref, wl_ref, bl_ref,
              cbt_ref, aw1t_ref, ab1_ref, aw2_ref, ab2_ref, awpt_ref, abp_ref,
              ow1t_ref, ob1_ref, ow2_ref, ob2_ref, owo_ref, obo_ref,
              g1_ref, lpr_ref,
              rep_ref, reot_ref, latt_ref, vqt_ref, quantt_ref):
    f32 = jnp.float32
    bf16 = jnp.bfloat16
    xot = ot_ref[...]
    a = a_ref[...]

    iota_v = lax.broadcasted_iota(jnp.int32, (ANUM, BB), 0)
    a_hott = jnp.where(iota_v == a, 1.0, 0.0).astype(f32)

    w1 = w1_ref[...]
    ht = _mm(w1[:, OBS:], a_hott) + _mm(w1[:, :OBS], xot) \
        + b1_ref[...][:, None]
    ht = jnp.maximum(ht, 0.0)
    h2t = jnp.maximum(_dx(w2t_ref[...], ht) + b2_ref[...][:, None],
                      0.0)
    latt = _mm(wl_ref[...], h2t) + bl_ref[...][:, None]
    latt_ref[...] = latt

    cbt = cbt_ref[...]
    md = -2.0 * jnp.repeat(jnp.tile(cbt, (VQ_DIM, 1)), LAT, axis=1) \
        * g1_ref[...]
    qmt = -0.125 * md
    cb2v = jnp.sum(0.25 * md * md, axis=0, keepdims=True)

    score = _dx(latt, md) + cb2v
    s = jnp.minimum(score, pltpu.roll(score, 64, 1))
    s = jnp.minimum(s, pltpu.roll(s, 32, 1))
    min_t = jnp.minimum(s, pltpu.roll(s, 16, 1))
    onehot = jnp.where(score == min_t, 1.0, 0.0)
    cnt = _mm(onehot, lpr_ref[...])
    first = jnp.where(cnt == 0.0, onehot, 0.0)
    qt = _dt(qmt, first)
    vqt = latt + (qt - latt)
    vqt_ref[...] = vqt
    quantt_ref[...] = qt

    vqbt = vqt.astype(bf16)
    xotb = xot.astype(bf16)
    aw1t = aw1t_ref[...]
    ha = _dx(vqbt, aw1t[:LAT].astype(bf16)) \
        + _dx(xotb, aw1t[LAT:].astype(bf16)) + ab1_ref[...][None, :]
    ha = jnp.maximum(ha, 0.0).astype(bf16)
    ha = jnp.maximum(_dt(ha, aw2_ref[...].astype(bf16))
                     + ab2_ref[...][None, :], 0.0).astype(bf16)
    rep_ref[...] = _mm(ha, awpt_ref[...].astype(bf16)) \
        + abp_ref[...][None, :]

    ho = jnp.maximum(_dx(vqbt, ow1t_ref[...].astype(bf16))
                     + ob1_ref[...][None, :], 0.0).astype(bf16)
    ho = jnp.maximum(_dt(ho, ow2_ref[...].astype(bf16))
                     + ob2_ref[...][None, :], 0.0).astype(bf16)
    reot_ref[...] = _dt(owo_ref[...].astype(bf16), ho) \
        + obo_ref[...][:, None]


def kernel(o, a, enc_w1, enc_b1, enc_w2, enc_b2, enc_wl, enc_bl, codebook,
           ad_w1, ad_b1, ad_w2, ad_b2, ad_wp, ad_bp,
           od_w1, od_b1, od_w2, od_b2, od_wo, od_bo):
    f32 = jnp.float32

    consts = [jnp.asarray(_G1), jnp.asarray(_LPR)]
    weights = [enc_w1, enc_b1, enc_w2.T, enc_b2, enc_wl, enc_bl,
               codebook.T, ad_w1.T, ad_b1, ad_w2, ad_b2, ad_wp.T,
               ad_bp, od_w1.T, od_b1, od_w2, od_b2, od_wo,
               od_bo] + consts

    def bcast_spec(arr):
        if arr.ndim == 1:
            return pl.BlockSpec(arr.shape, lambda i: (0,))
        return pl.BlockSpec(arr.shape, lambda i: (0, 0))

    in_specs = [
        pl.BlockSpec((OBS, BB), lambda i: (0, i)),
        pl.BlockSpec((1, BB), lambda i: (0, i)),
    ] + [bcast_spec(w) for w in weights]

    out_specs = (
        pl.BlockSpec((BB, ANUM), lambda i: (i, 0)),
        pl.BlockSpec((OBS, BB), lambda i: (0, i)),
        pl.BlockSpec((LAT, BB), lambda i: (0, i)),
        pl.BlockSpec((LAT, BB), lambda i: (0, i)),
        pl.BlockSpec((LAT, BB), lambda i: (0, i)),
    )
    out_shape = (
        jax.ShapeDtypeStruct((B, ANUM), f32),
        jax.ShapeDtypeStruct((OBS, B), f32),
        jax.ShapeDtypeStruct((LAT, B), f32),
        jax.ShapeDtypeStruct((LAT, B), f32),
        jax.ShapeDtypeStruct((LAT, B), f32),
    )

    re_p, re_ot, latentt, vq_latentt, quantizedt = pl.pallas_call(
        _vq_block,
        grid=(B // BB,),
        in_specs=in_specs,
        out_specs=out_specs,
        out_shape=out_shape,
    )(o.T, a.reshape(1, B), *weights)
    return (re_p, re_ot.T, latentt.T, vq_latentt.T, quantizedt.T)
